# baseline (speedup 1.0000x reference)
import functools

import jax
import jax.numpy as jnp
from jax import lax
from jax.experimental import pallas as pl
from jax.experimental.pallas import tpu as pltpu
from jax.experimental.pallas import tpu_sc as plsc

D = 64
NC, NS = 2, 16
NW = NC * NS
B_TOTAL = 4096 * 200
PER_W = B_TOTAL // NW
CHUNK = 512
STEPS = PER_W // CHUNK

_mesh = plsc.VectorSubcoreMesh(core_axis_name="c", subcore_axis_name="s")


@functools.partial(
    pl.kernel,
    out_type=jax.ShapeDtypeStruct((B_TOTAL, D), jnp.float32),
    mesh=_mesh,
    scratch_types=[
        pltpu.VMEM((CHUNK,), jnp.int32),
        pltpu.VMEM((CHUNK, D), jnp.float32),
        pltpu.SemaphoreType.DMA,
    ],
    compiler_params=pltpu.CompilerParams(use_tc_tiling_on_sc=False),
)
def _emb_lookup(x_hbm, w_hbm, out_hbm, idx_v, rows_v, sem):
    wid = lax.axis_index("s") * NC + lax.axis_index("c")
    base = wid * PER_W

    @pl.loop(0, STEPS)
    def _chunk(i):
        off = base + i * CHUNK
        pltpu.sync_copy(x_hbm.at[pl.ds(off, CHUNK)], idx_v)
        pltpu.async_copy(w_hbm.at[idx_v], rows_v, sem).wait()
        pltpu.sync_copy(rows_v, out_hbm.at[pl.ds(off, CHUNK)])


def kernel(x, weight):
    B, H = x.shape
    flat = x.reshape(B * H)
    out = _emb_lookup(flat, weight)
    return out.reshape(B, H, D)


# speedup vs baseline: 1.0426x; 1.0426x over previous
import functools

import jax
import jax.numpy as jnp
from jax import lax
from jax.experimental import pallas as pl
from jax.experimental.pallas import tpu as pltpu
from jax.experimental.pallas import tpu_sc as plsc

D = 64
NC, NS = 2, 16
NW = NC * NS
B_TOTAL = 4096 * 200
PER_W = B_TOTAL // NW
CHUNK = 400
STEPS = PER_W // CHUNK
R = 4
GROUPS = (STEPS - 2 * R) // R

_mesh = plsc.VectorSubcoreMesh(core_axis_name="c", subcore_axis_name="s")


@functools.partial(
    pl.kernel,
    out_type=jax.ShapeDtypeStruct((B_TOTAL, D), jnp.float32),
    mesh=_mesh,
    scratch_types=[
        pltpu.VMEM((R, CHUNK), jnp.int32),
        pltpu.VMEM((R, CHUNK, D), jnp.float32),
        pltpu.SemaphoreType.DMA((R,)),
        pltpu.SemaphoreType.DMA((R,)),
        pltpu.SemaphoreType.DMA((R,)),
    ],
    compiler_params=pltpu.CompilerParams(use_tc_tiling_on_sc=False),
)
def _emb_lookup(x_hbm, w_hbm, out_hbm, idx_v, rows_v, si, sg, so):
    wid = lax.axis_index("s") * NC + lax.axis_index("c")
    base = wid * PER_W

    def start_idx(b, chunk):
        off = base + chunk * CHUNK
        pltpu.async_copy(x_hbm.at[pl.ds(off, CHUNK)], idx_v.at[b], si.at[b])

    def wait_idx(b):
        pltpu.make_async_copy(
            x_hbm.at[pl.ds(base, CHUNK)], idx_v.at[b], si.at[b]).wait()

    def start_gather(b):
        pltpu.async_copy(w_hbm.at[idx_v.at[b]], rows_v.at[b], sg.at[b])

    def wait_gather(b):
        pltpu.make_async_copy(
            w_hbm.at[idx_v.at[b]], rows_v.at[b], sg.at[b]).wait()

    def start_out(b, chunk):
        off = base + chunk * CHUNK
        pltpu.async_copy(rows_v.at[b], out_hbm.at[pl.ds(off, CHUNK)], so.at[b])

    def wait_out(b):
        pltpu.make_async_copy(
            rows_v.at[b], out_hbm.at[pl.ds(base, CHUNK)], so.at[b]).wait()

    for b in range(R):
        start_idx(b, b)
    for b in range(R):
        wait_idx(b)
        start_gather(b)
    for b in range(R - 1):
        wait_gather(b)
        start_out(b, b)
        start_idx(b, b + R)

    @pl.loop(0, GROUPS)
    def _grp(g):
        for r in range(R):
            i = R + g * R + r
            b = r
            bp = (r + R - 1) % R
            wait_gather(bp)
            start_out(bp, i - 1)
            start_idx(bp, i - 1 + R)
            wait_idx(b)
            wait_out(b)
            start_gather(b)

    for i in range(STEPS - R, STEPS):
        b = i % R
        bp = (i - 1) % R
        wait_gather(bp)
        start_out(bp, i - 1)
        if i - 1 + R < STEPS:
            start_idx(bp, i - 1 + R)
        wait_idx(b)
        wait_out(b)
        start_gather(b)
    b_last = (STEPS - 1) % R
    wait_gather(b_last)
    start_out(b_last, STEPS - 1)
    for b in range(R):
        wait_out(b)


def kernel(x, weight):
    B, H = x.shape
    flat = x.reshape(B * H)
    out = _emb_lookup(flat, weight)
    return out.reshape(B, H, D)
